# baseline (speedup 1.0000x reference)
import jax
import jax.numpy as jnp
from jax import lax
from jax.experimental import pallas as pl

N = 128
D = 128
SIGMA = 2.5
PB = 8


def _adj(c, ct):
    d2 = jnp.zeros((N, N), jnp.float32)
    for k in range(3):
        diff = c[:, k:k + 1] - ct[k:k + 1, :]
        d2 = d2 + diff * diff
    dist = jnp.sqrt(d2 + 1e-12)
    a = jnp.exp(dist * (-1.0 / SIGMA))
    ii = lax.broadcasted_iota(jnp.int32, (N, N), 0)
    jj = lax.broadcasted_iota(jnp.int32, (N, N), 1)
    a = jnp.where(ii == jj, 1.0, a)
    deg = jnp.sum(a, axis=1, keepdims=True)
    dinv = lax.rsqrt(deg)
    return a, dinv


def _body(co_ref, cto_ref, xo_ref, cm_ref, ctm_ref, xm_ref,
          w1_ref, b1_ref, w2_ref, b2_ref,
          wt1a_ref, wt1b_ref, bt1_ref, wt2_ref, bt2_ref, out_ref):
    w1 = w1_ref[...]
    b1 = b1_ref[...]
    w2 = w2_ref[...]
    b2 = b2_ref[...]

    NG = 2 * PB
    feats = [xo_ref, xm_ref]
    cs = [co_ref, cm_ref]
    cts = [cto_ref, ctm_ref]

    def gref(i):
        return i % 2, i // 2

    y = [None] * NG
    for i in range(NG):
        r, p = gref(i)
        y[i] = jnp.dot(feats[r][p], w1, preferred_element_type=jnp.float32)

    adj = [None] * NG
    for i in range(NG):
        r, p = gref(i)
        adj[i] = _adj(cs[r][p], cts[r][p])

    h = [None] * NG
    for i in range(NG):
        a, dinv = adj[i]
        z = dinv * jnp.dot(a, dinv * y[i], preferred_element_type=jnp.float32)
        h[i] = jnp.maximum(z + b1, 0.0)

    emb = [None] * NG
    for i in range(NG):
        a, dinv = adj[i]
        y2 = jnp.dot(h[i], w2, preferred_element_type=jnp.float32)
        z2 = dinv * jnp.dot(a, dinv * y2, preferred_element_type=jnp.float32)
        h2 = jnp.maximum(z2 + b2, 0.0)
        emb[i] = jnp.max(h2, axis=0, keepdims=True)

    emb_o = jnp.concatenate([emb[2 * p] for p in range(PB)], axis=0)
    emb_m = jnp.concatenate([emb[2 * p + 1] for p in range(PB)], axis=0)
    hrow = jnp.maximum(
        jnp.dot(emb_o, wt1a_ref[...], preferred_element_type=jnp.float32)
        + jnp.dot(emb_m, wt1b_ref[...], preferred_element_type=jnp.float32)
        + bt1_ref[...], 0.0)
    logit = (jnp.dot(hrow, wt2_ref[...], preferred_element_type=jnp.float32)
             + bt2_ref[...])
    out_ref[...] = logit[:, :, None]


def kernel(coords_orig, feats_orig, coords_mut, feats_mut,
           W1, b1, W2, b2, Wt1, bt1, Wt2, bt2):
    B = coords_orig.shape[0]
    cto = jnp.swapaxes(coords_orig, 1, 2)
    ctm = jnp.swapaxes(coords_mut, 1, 2)

    gb = lambda b: (b, 0, 0)
    cb = lambda b: (0, 0)
    out = pl.pallas_call(
        _body,
        grid=(B // PB,),
        in_specs=[
            pl.BlockSpec((PB, N, 3), gb),
            pl.BlockSpec((PB, 3, N), gb),
            pl.BlockSpec((PB, N, D), gb),
            pl.BlockSpec((PB, N, 3), gb),
            pl.BlockSpec((PB, 3, N), gb),
            pl.BlockSpec((PB, N, D), gb),
            pl.BlockSpec((D, D), cb),
            pl.BlockSpec((1, D), cb),
            pl.BlockSpec((D, D), cb),
            pl.BlockSpec((1, D), cb),
            pl.BlockSpec((D, D), cb),
            pl.BlockSpec((D, D), cb),
            pl.BlockSpec((1, D), cb),
            pl.BlockSpec((D, 1), cb),
            pl.BlockSpec((1, 1), cb),
        ],
        out_specs=pl.BlockSpec((PB, 1, 1), gb),
        out_shape=jax.ShapeDtypeStruct((B, 1, 1), jnp.float32),
    )(coords_orig, cto, feats_orig, coords_mut, ctm, feats_mut,
      W1, b1.reshape(1, D), W2, b2.reshape(1, D),
      Wt1[:D], Wt1[D:], bt1.reshape(1, D), Wt2, bt2.reshape(1, 1))
    return out.reshape(B, 1)


# speedup vs baseline: 1.1895x; 1.1895x over previous
import jax
import jax.numpy as jnp
from jax import lax
from jax.experimental import pallas as pl

N = 128
D = 128
SIGMA = 2.5
PB = 8


def _adj(c):
    ct = c.T
    d2 = jnp.zeros((N, N), jnp.float32)
    for k in range(3):
        diff = c[:, k:k + 1] - ct[k:k + 1, :]
        d2 = d2 + diff * diff
    dist = jnp.sqrt(d2 + 1e-12)
    a = jnp.exp(dist * (-1.0 / SIGMA))
    deg = jnp.sum(a, axis=1, keepdims=True)
    dinv = lax.rsqrt(deg)
    return a, dinv


def _body(co_ref, xo_ref, cm_ref, xm_ref,
          w1_ref, b1_ref, w2_ref, b2_ref,
          wt1_ref, bt1_ref, wt2_ref, bt2_ref, out_ref):
    w1 = w1_ref[...]
    b1 = b1_ref[...]
    w2 = w2_ref[...]
    b2 = b2_ref[...]

    NG = 2 * PB
    feats = [xo_ref, xm_ref]
    cs = [co_ref, cm_ref]

    def gref(i):
        return i % 2, i // 2

    y = [None] * NG
    for i in range(NG):
        r, p = gref(i)
        y[i] = jnp.dot(feats[r][p], w1, preferred_element_type=jnp.float32)

    adj = [None] * NG
    for i in range(NG):
        r, p = gref(i)
        adj[i] = _adj(cs[r][p])

    h = [None] * NG
    for i in range(NG):
        a, dinv = adj[i]
        z = dinv * jnp.dot(a, dinv * y[i], preferred_element_type=jnp.float32)
        h[i] = jnp.maximum(z + b1, 0.0)

    emb = [None] * NG
    for i in range(NG):
        a, dinv = adj[i]
        y2 = jnp.dot(h[i], w2, preferred_element_type=jnp.float32)
        z2 = dinv * jnp.dot(a, dinv * y2, preferred_element_type=jnp.float32)
        h2 = jnp.maximum(z2 + b2, 0.0)
        emb[i] = jnp.max(h2, axis=0, keepdims=True)

    emb_o = jnp.concatenate([emb[2 * p] for p in range(PB)], axis=0)
    emb_m = jnp.concatenate([emb[2 * p + 1] for p in range(PB)], axis=0)
    hrow = jnp.maximum(
        jnp.dot(emb_o, wt1_ref[:D], preferred_element_type=jnp.float32)
        + jnp.dot(emb_m, wt1_ref[D:], preferred_element_type=jnp.float32)
        + bt1_ref[...], 0.0)
    out_ref[...] = (
        jnp.dot(hrow, wt2_ref[...], preferred_element_type=jnp.float32)
        + bt2_ref[...])


def kernel(coords_orig, feats_orig, coords_mut, feats_mut,
           W1, b1, W2, b2, Wt1, bt1, Wt2, bt2):
    B = coords_orig.shape[0]
    gb = lambda b: (b, 0, 0)
    cb2 = lambda b: (0, 0)
    cb1 = lambda b: (0,)
    out = pl.pallas_call(
        _body,
        grid=(B // PB,),
        in_specs=[
            pl.BlockSpec((PB, N, 3), gb),
            pl.BlockSpec((PB, N, D), gb),
            pl.BlockSpec((PB, N, 3), gb),
            pl.BlockSpec((PB, N, D), gb),
            pl.BlockSpec((D, D), cb2),
            pl.BlockSpec((D,), cb1),
            pl.BlockSpec((D, D), cb2),
            pl.BlockSpec((D,), cb1),
            pl.BlockSpec((2 * D, D), cb2),
            pl.BlockSpec((D,), cb1),
            pl.BlockSpec((D, 1), cb2),
            pl.BlockSpec((1,), cb1),
        ],
        out_specs=pl.BlockSpec((PB, 1), lambda b: (b, 0)),
        out_shape=jax.ShapeDtypeStruct((B, 1), jnp.float32),
    )(coords_orig, feats_orig, coords_mut, feats_mut,
      W1, b1, W2, b2, Wt1, bt1, Wt2, bt2)
    return out


# speedup vs baseline: 1.1989x; 1.0079x over previous
import jax
import jax.numpy as jnp
from jax import lax
from jax.experimental import pallas as pl

N = 128
D = 128
SIGMA = 2.5
PB = 8


def _adj(c):
    ct = c.T
    d2 = jnp.zeros((N, N), jnp.float32)
    for k in range(3):
        diff = c[:, k:k + 1] - ct[k:k + 1, :]
        d2 = d2 + diff * diff
    dist = jnp.sqrt(d2 + 1e-12)
    a = jnp.exp(dist * (-1.0 / SIGMA))
    deg_c = jnp.sum(a, axis=1, keepdims=True)
    deg_r = jnp.sum(a, axis=0, keepdims=True)
    return a * lax.rsqrt(deg_c) * lax.rsqrt(deg_r)


def _body(co_ref, xo_ref, cm_ref, xm_ref,
          w1_ref, b1_ref, w2_ref, b2_ref,
          wt1_ref, bt1_ref, wt2_ref, bt2_ref, out_ref):
    w1 = w1_ref[...]
    b1 = b1_ref[...]
    w2 = w2_ref[...]
    b2 = b2_ref[...]

    NG = 2 * PB
    feats = [xo_ref, xm_ref]
    cs = [co_ref, cm_ref]

    def gref(i):
        return i % 2, i // 2

    adj = [None] * NG
    for i in range(NG):
        r, p = gref(i)
        adj[i] = _adj(cs[r][p])

    h = [None] * NG
    for i in range(NG):
        r, p = gref(i)
        g1 = jnp.dot(adj[i], feats[r][p], preferred_element_type=jnp.float32)
        z = jnp.dot(g1, w1, preferred_element_type=jnp.float32)
        h[i] = jnp.maximum(z + b1, 0.0)

    emb = [None] * NG
    for i in range(NG):
        g2 = jnp.dot(adj[i], h[i], preferred_element_type=jnp.float32)
        z2 = jnp.dot(g2, w2, preferred_element_type=jnp.float32)
        h2 = jnp.maximum(z2 + b2, 0.0)
        emb[i] = jnp.max(h2, axis=0, keepdims=True)

    emb_o = jnp.concatenate([emb[2 * p] for p in range(PB)], axis=0)
    emb_m = jnp.concatenate([emb[2 * p + 1] for p in range(PB)], axis=0)
    hrow = jnp.maximum(
        jnp.dot(emb_o, wt1_ref[:D], preferred_element_type=jnp.float32)
        + jnp.dot(emb_m, wt1_ref[D:], preferred_element_type=jnp.float32)
        + bt1_ref[...], 0.0)
    out_ref[...] = (
        jnp.dot(hrow, wt2_ref[...], preferred_element_type=jnp.float32)
        + bt2_ref[...])


def kernel(coords_orig, feats_orig, coords_mut, feats_mut,
           W1, b1, W2, b2, Wt1, bt1, Wt2, bt2):
    B = coords_orig.shape[0]
    gb = lambda b: (b, 0, 0)
    cb2 = lambda b: (0, 0)
    cb1 = lambda b: (0,)
    out = pl.pallas_call(
        _body,
        grid=(B // PB,),
        in_specs=[
            pl.BlockSpec((PB, N, 3), gb),
            pl.BlockSpec((PB, N, D), gb),
            pl.BlockSpec((PB, N, 3), gb),
            pl.BlockSpec((PB, N, D), gb),
            pl.BlockSpec((D, D), cb2),
            pl.BlockSpec((D,), cb1),
            pl.BlockSpec((D, D), cb2),
            pl.BlockSpec((D,), cb1),
            pl.BlockSpec((2 * D, D), cb2),
            pl.BlockSpec((D,), cb1),
            pl.BlockSpec((D, 1), cb2),
            pl.BlockSpec((1,), cb1),
        ],
        out_specs=pl.BlockSpec((PB, 1), lambda b: (b, 0)),
        out_shape=jax.ShapeDtypeStruct((B, 1), jnp.float32),
    )(coords_orig, feats_orig, coords_mut, feats_mut,
      W1, b1, W2, b2, Wt1, bt1, Wt2, bt2)
    return out
